# baseline (speedup 1.0000x reference)
import functools

import jax
import jax.numpy as jnp
from jax import lax
from jax.experimental import pallas as pl
from jax.experimental.pallas import tpu as pltpu
from jax.experimental.pallas import tpu_sc as plsc

_NC = 2
_NS = 16
_NW = _NC * _NS
_CP = 32
_NBUF = 4


def _proj_body(t0_ref, t1_ref, t2_ref, t3_ref, w_ref, o_ref):
    parts = [
        jax.lax.dot_general(
            t_ref[...].astype(jnp.bfloat16), w_ref[...],
            (((0,), (0,)), ((), ())), preferred_element_type=jnp.float32)
        for t_ref in (t0_ref, t1_ref, t2_ref, t3_ref)
    ]
    o_ref[...] = jnp.concatenate(parts, axis=1)


def _make_sc(B, L):
    LP = L + 2
    bags_per_w = B // _NW
    mesh = plsc.VectorSubcoreMesh(core_axis_name="c", subcore_axis_name="s")

    steps = bags_per_w // 2

    @functools.partial(
        pl.kernel,
        out_type=jax.ShapeDtypeStruct((B, _CP), jnp.float32),
        mesh=mesh,
        compiler_params=pltpu.CompilerParams(use_tc_tiling_on_sc=False),
        scratch_types=[
            pltpu.VMEM((steps * 2 * LP,), jnp.int32),
            pltpu.VMEM((_NBUF, 2 * LP), jnp.int32),
            pltpu.VMEM((_NBUF, 2 * LP, _CP), jnp.float32),
            pltpu.VMEM((bags_per_w, _CP), jnp.float32),
            pltpu.VMEM((_CP,), jnp.float32),
            pltpu.SemaphoreType.DMA((_NBUF,)),
        ],
    )
    def sc_fn(idx_hbm, proj_hbm, b_hbm, out_hbm, idx_v, slot_v, rows_v,
              out_v, b_v, sem):
        wid = lax.axis_index("s") * _NC + lax.axis_index("c")
        base = wid * bags_per_w
        pltpu.sync_copy(idx_hbm.at[pl.ds(base * LP, bags_per_w * LP)],
                        idx_v)
        pltpu.sync_copy(b_hbm, b_v)
        b0 = b_v[0:16]
        b1 = b_v[16:32]
        inv = 1.0 / L
        offs = list(range(0, 2 * LP - 15, 16)) + [2 * LP - 16]

        def start(j, buf):
            for o in offs:
                slot_v[buf, o:o + 16] = idx_v[pl.ds(j * 2 * LP + o, 16)]
            pltpu.async_copy(proj_hbm.at[slot_v.at[buf]],
                             rows_v.at[buf], sem.at[buf])

        for j in range(_NBUF - 1):
            start(j, j)

        def group(g, carry):
            for bslot in range(_NBUF):
                j = g * _NBUF + bslot
                pltpu.make_async_copy(
                    proj_hbm.at[slot_v.at[bslot]],
                    rows_v.at[bslot], sem.at[bslot]).wait()

                @pl.when(j + _NBUF - 1 < steps)
                def _():
                    start(j + _NBUF - 1, (bslot + _NBUF - 1) % _NBUF)

                def inner(l, acc):
                    a0, a1, c0, c1 = acc
                    return (a0 + rows_v[bslot, l, 0:16],
                            a1 + rows_v[bslot, l, 16:32],
                            c0 + rows_v[bslot, LP + l, 0:16],
                            c1 + rows_v[bslot, LP + l, 16:32])

                z = jnp.zeros((16,), jnp.float32)
                a0, a1, c0, c1 = lax.fori_loop(0, L, inner, (z, z, z, z))
                out_v[2 * j, 0:16] = a0 * inv + b0
                out_v[2 * j, 16:32] = a1 * inv + b1
                out_v[2 * j + 1, 0:16] = c0 * inv + b0
                out_v[2 * j + 1, 16:32] = c1 * inv + b1
            return carry

        lax.fori_loop(0, steps // _NBUF, group, 0)
        pltpu.sync_copy(out_v, out_hbm.at[pl.ds(base, bags_per_w)])

    return sc_fn


def kernel(text, table, W, b):
    V, D = table.shape
    B, L = text.shape
    C = W.shape[1]
    w_pad = jnp.pad(W, ((0, 0), (0, _CP - C))).astype(jnp.bfloat16)
    b_pad = jnp.pad(b, (0, _CP - C))

    vb = 3200
    S = 25600
    nb = S // vb
    tT = table.T
    proj_packed = pl.pallas_call(
        _proj_body,
        grid=(nb,),
        in_specs=[pl.BlockSpec((D, vb),
                               lambda i, k=k: (0, i + nb * k))
                  for k in range(4)] +
                 [pl.BlockSpec((D, _CP), lambda i: (0, 0))],
        out_specs=pl.BlockSpec((vb, 128), lambda i: (i, 0)),
        out_shape=jax.ShapeDtypeStruct((S, 128), jnp.float32),
    )(tT, tT, tT, tT, w_pad)

    proj32 = proj_packed.reshape(4 * S, _CP)
    idx_map = (text % S) * 4 + text // S
    idx_flat = jnp.pad(idx_map, ((0, 0), (0, 2))).reshape(B * (L + 2))
    pooled = _make_sc(B, L)(idx_flat, proj32, b_pad)
    return pooled[:, :C]


# speedup vs baseline: 1.2771x; 1.2771x over previous
import functools

import jax
import jax.numpy as jnp
from jax import lax
from jax.experimental import pallas as pl
from jax.experimental.pallas import tpu as pltpu
from jax.experimental.pallas import tpu_sc as plsc

_NC = 2
_NS = 16
_NW = _NC * _NS
_CP = 128
_NBUF = 4


def _proj_body(tT_ref, w_ref, o_ref):
    o_ref[...] = jax.lax.dot_general(
        tT_ref[...].astype(jnp.bfloat16), w_ref[...],
        (((0,), (0,)), ((), ())), preferred_element_type=jnp.float32)


def _make_sc(B, L):
    bags_per_w = B // _NW
    mesh = plsc.VectorSubcoreMesh(core_axis_name="c", subcore_axis_name="s")

    steps = bags_per_w // 2

    @functools.partial(
        pl.kernel,
        out_type=jax.ShapeDtypeStruct((B, _CP), jnp.float32),
        mesh=mesh,
        compiler_params=pltpu.CompilerParams(use_tc_tiling_on_sc=True),
        scratch_types=[
            pltpu.VMEM((steps, 2 * L), jnp.int32),
            pltpu.VMEM((_NBUF, 2 * L, _CP), jnp.float32),
            pltpu.VMEM((bags_per_w, _CP), jnp.float32),
            pltpu.VMEM((32,), jnp.float32),
            pltpu.SemaphoreType.DMA((_NBUF,)),
        ],
    )
    def sc_fn(idx_hbm, proj_hbm, b_hbm, out_hbm, idx_v, rows_v, out_v, b_v,
              sem):
        wid = lax.axis_index("s") * _NC + lax.axis_index("c")
        base = wid * bags_per_w
        pltpu.sync_copy(idx_hbm.at[pl.ds(wid * steps, steps)], idx_v)
        pltpu.sync_copy(b_hbm, b_v)
        b0 = b_v[0:16]
        b1 = b_v[16:32]
        inv = 1.0 / L

        def start(j, buf):
            pltpu.async_copy(proj_hbm.at[idx_v.at[j]],
                             rows_v.at[buf], sem.at[buf])

        for j in range(_NBUF - 1):
            start(j, j)

        def group(g, carry):
            for bslot in range(_NBUF):
                j = g * _NBUF + bslot
                pltpu.make_async_copy(proj_hbm.at[idx_v.at[j]],
                                      rows_v.at[bslot],
                                      sem.at[bslot]).wait()

                @pl.when(j + _NBUF - 1 < steps)
                def _():
                    start(j + _NBUF - 1, (bslot + _NBUF - 1) % _NBUF)

                def inner(l, acc):
                    a0, a1, c0, c1 = acc
                    return (a0 + rows_v[bslot, l, 0:16],
                            a1 + rows_v[bslot, l, 16:32],
                            c0 + rows_v[bslot, L + l, 0:16],
                            c1 + rows_v[bslot, L + l, 16:32])

                z = jnp.zeros((16,), jnp.float32)
                a0, a1, c0, c1 = lax.fori_loop(0, L, inner, (z, z, z, z))
                out_v[2 * j, 0:16] = a0 * inv + b0
                out_v[2 * j, 16:32] = a1 * inv + b1
                out_v[2 * j + 1, 0:16] = c0 * inv + b0
                out_v[2 * j + 1, 16:32] = c1 * inv + b1
            return carry

        lax.fori_loop(0, steps // _NBUF, group, 0)
        pltpu.sync_copy(out_v, out_hbm.at[pl.ds(base, bags_per_w)])

    return sc_fn


def kernel(text, table, W, b):
    V, D = table.shape
    B, L = text.shape
    C = W.shape[1]
    w_pad = jnp.pad(W, ((0, 0), (0, _CP - C))).astype(jnp.bfloat16)
    b_pad = jnp.pad(b, (0, 32 - C))

    vb = 4096
    proj = pl.pallas_call(
        _proj_body,
        grid=(pl.cdiv(V, vb),),
        in_specs=[pl.BlockSpec((D, vb), lambda i: (0, i)),
                  pl.BlockSpec((D, _CP), lambda i: (0, 0))],
        out_specs=pl.BlockSpec((vb, _CP), lambda i: (i, 0)),
        out_shape=jax.ShapeDtypeStruct((V, _CP), jnp.float32),
    )(table.T, w_pad)

    idx2 = text.reshape(B // 2, 2 * L)
    pooled = _make_sc(B, L)(idx2, proj, b_pad)
    return pooled[:, :C]


# speedup vs baseline: 1.3113x; 1.0268x over previous
import functools

import jax
import jax.numpy as jnp
from jax import lax
from jax.experimental import pallas as pl
from jax.experimental.pallas import tpu as pltpu
from jax.experimental.pallas import tpu_sc as plsc

_NC = 2
_NS = 16
_NW = _NC * _NS
_CP = 128
_NBUF = 4


def _proj_body(tT_ref, w_ref, o_ref):
    o_ref[...] = jax.lax.dot_general(
        tT_ref[...].astype(jnp.bfloat16), w_ref[...],
        (((0,), (0,)), ((), ())), preferred_element_type=jnp.float32)


def _make_sc(B, L):
    bags_per_w = B // _NW
    mesh = plsc.VectorSubcoreMesh(core_axis_name="c", subcore_axis_name="s")

    steps = bags_per_w // 2

    @functools.partial(
        pl.kernel,
        out_type=jax.ShapeDtypeStruct((B, _CP), jnp.float32),
        mesh=mesh,
        compiler_params=pltpu.CompilerParams(use_tc_tiling_on_sc=True),
        scratch_types=[
            pltpu.VMEM((steps, 2 * L), jnp.int32),
            pltpu.VMEM((_NBUF, 2 * L, _CP), jnp.float32),
            pltpu.VMEM((bags_per_w, _CP), jnp.float32),
            pltpu.VMEM((32,), jnp.float32),
            pltpu.SemaphoreType.DMA((_NBUF,)),
        ],
    )
    def sc_fn(idx_hbm, proj_hbm, b_hbm, out_hbm, idx_v, rows_v, out_v, b_v,
              sem):
        wid = lax.axis_index("s") * _NC + lax.axis_index("c")
        base = wid * bags_per_w
        pltpu.sync_copy(idx_hbm.at[pl.ds(wid * steps, steps)], idx_v)
        pltpu.sync_copy(b_hbm, b_v)
        b0 = b_v[0:16]
        b1 = b_v[16:32]
        inv = 1.0 / L

        def start(j, buf):
            pltpu.async_copy(proj_hbm.at[idx_v.at[j]],
                             rows_v.at[buf], sem.at[buf])

        for j in range(_NBUF - 1):
            start(j, j)

        def group(g, carry):
            for bslot in range(_NBUF):
                j = g * _NBUF + bslot
                pltpu.make_async_copy(proj_hbm.at[idx_v.at[j]],
                                      rows_v.at[bslot],
                                      sem.at[bslot]).wait()

                @pl.when(j + _NBUF - 1 < steps)
                def _():
                    start(j + _NBUF - 1, (bslot + _NBUF - 1) % _NBUF)

                def inner(l, acc):
                    a0, a1, c0, c1 = acc
                    return (a0 + rows_v[bslot, l, 0:16],
                            a1 + rows_v[bslot, l, 16:32],
                            c0 + rows_v[bslot, L + l, 0:16],
                            c1 + rows_v[bslot, L + l, 16:32])

                z = jnp.zeros((16,), jnp.float32)
                a0, a1, c0, c1 = lax.fori_loop(0, L, inner, (z, z, z, z))
                out_v[2 * j, 0:16] = a0 * inv + b0
                out_v[2 * j, 16:32] = a1 * inv + b1
                out_v[2 * j + 1, 0:16] = c0 * inv + b0
                out_v[2 * j + 1, 16:32] = c1 * inv + b1
            return carry

        lax.fori_loop(0, steps // _NBUF, group, 0)
        pltpu.sync_copy(out_v, out_hbm.at[pl.ds(base, bags_per_w)])

    return sc_fn


def kernel(text, table, W, b):
    V, D = table.shape
    B, L = text.shape
    C = W.shape[1]
    w_pad = jnp.pad(W, ((0, 0), (0, _CP - C))).astype(jnp.bfloat16)
    b_pad = jnp.pad(b, (0, 32 - C))

    vb = 8192
    proj = pl.pallas_call(
        _proj_body,
        grid=(pl.cdiv(V, vb),),
        in_specs=[pl.BlockSpec((D, vb), lambda i: (0, i)),
                  pl.BlockSpec((D, _CP), lambda i: (0, 0))],
        out_specs=pl.BlockSpec((vb, _CP), lambda i: (i, 0)),
        out_shape=jax.ShapeDtypeStruct((V, _CP), jnp.float32),
    )(table.T, w_pad)

    idx2 = text.reshape(B // 2, 2 * L)
    pooled = _make_sc(B, L)(idx2, proj, b_pad)
    return pooled[:, :C]
